# Initial kernel scaffold; baseline (speedup 1.0000x reference)
import jax
import jax.numpy as jnp
from jax import lax
from jax.experimental import pallas as pl
from jax.experimental.pallas import tpu as pltpu
from jax.experimental.pallas import tpu_sc as plsc

_NUM_FREQ = 6
_NUM_RAW = 16384
_SEG = 2048
_NUM_VIDS = 8
_TCH = 1 + 2 * _NUM_FREQ
_D = 128
_B = 262144
_NC, _NS = 2, 16
_NW = _NC * _NS
_BPW = _B // _NW
_CH = 128
_NCHUNK = _BPW // _CH


def _table_body(w1t_ref, b1_ref, inst_ref, out_ref):
    row = lax.broadcasted_iota(jnp.float32, (_SEG, 1), 0)
    t = (row / float(_SEG - 1)) * 2.0 - 1.0
    acc = b1_ref[...] + inst_ref[0] + t * w1t_ref[0:1, :]
    for k in range(_NUM_FREQ):
        sig = t * (2.0 ** k)
        acc = acc + jnp.sin(sig) * w1t_ref[2 * k + 1:2 * k + 2, :]
        acc = acc + jnp.cos(sig) * w1t_ref[2 * k + 2:2 * k + 3, :]
    out_ref[...] = acc


def _gather_body(table_hbm, idx_hbm, out_hbm, idx_v, buf0, buf1, sem0, sem1):
    wid = lax.axis_index("s") * _NC + lax.axis_index("c")
    base = wid * _BPW
    pltpu.sync_copy(idx_hbm.at[wid], idx_v)
    bufs, sems = (buf0, buf1), (sem0, sem1)
    pltpu.async_copy(table_hbm.at[idx_v.at[0]], buf0, sem0)
    pltpu.async_copy(table_hbm.at[idx_v.at[1]], buf1, sem1)

    @pl.loop(0, _NCHUNK - 2, step=2)
    def _steady(j):
        for b in range(2):
            jj = j + b
            pltpu.make_async_copy(table_hbm.at[idx_v.at[jj]], bufs[b], sems[b]).wait()
            pltpu.sync_copy(bufs[b], out_hbm.at[pl.ds(base + jj * _CH, _CH)])
            pltpu.async_copy(table_hbm.at[idx_v.at[jj + 2]], bufs[b], sems[b])

    for b in range(2):
        jj = _NCHUNK - 2 + b
        pltpu.make_async_copy(table_hbm.at[idx_v.at[jj]], bufs[b], sems[b]).wait()
        pltpu.sync_copy(bufs[b], out_hbm.at[pl.ds(base + jj * _CH, _CH)])


def kernel(frame_id, W1, b1, inst_table):
    w1t = W1.T
    b1_2d = b1.reshape(1, _D)
    inst3 = inst_table.reshape(_NUM_VIDS, 1, _D)

    table = pl.pallas_call(
        _table_body,
        grid=(_NUM_VIDS,),
        in_specs=[
            pl.BlockSpec((_TCH, _D), lambda i: (0, 0)),
            pl.BlockSpec((1, _D), lambda i: (0, 0)),
            pl.BlockSpec((1, 1, _D), lambda i: (i, 0, 0)),
        ],
        out_specs=pl.BlockSpec((_SEG, _D), lambda i: (i, 0)),
        out_shape=jax.ShapeDtypeStruct((_NUM_RAW, _D), jnp.float32),
    )(w1t, b1_2d, inst3)

    idx = frame_id.reshape(_NW, _NCHUNK, _CH)
    mesh = plsc.VectorSubcoreMesh(core_axis_name="c", subcore_axis_name="s",
                                  num_cores=_NC, num_subcores=_NS)
    gather = pl.kernel(
        _gather_body,
        out_type=jax.ShapeDtypeStruct((_B, _D), jnp.float32),
        mesh=mesh,
        scratch_types=[
            pltpu.VMEM((_NCHUNK, _CH), jnp.int32),
            pltpu.VMEM((_CH, _D), jnp.float32),
            pltpu.VMEM((_CH, _D), jnp.float32),
            pltpu.SemaphoreType.DMA,
            pltpu.SemaphoreType.DMA,
        ],
    )
    return gather(table, idx)


# speedup vs baseline: 20.6115x; 20.6115x over previous
import jax
import jax.numpy as jnp
from jax import lax
from jax.experimental import pallas as pl
from jax.experimental.pallas import tpu as pltpu
from jax.experimental.pallas import tpu_sc as plsc

_NUM_FREQ = 6
_NUM_RAW = 16384
_SEG = 2048
_NUM_VIDS = 8
_TCH = 1 + 2 * _NUM_FREQ
_D = 128
_B = 262144
_NC, _NS = 2, 16
_NW = _NC * _NS
_BPW = _B // _NW
_CH = 128
_NCHUNK = _BPW // _CH


def _table_body(w1t_ref, b1_ref, inst_ref, out_ref):
    row = lax.broadcasted_iota(jnp.int32, (_SEG, 1), 0).astype(jnp.float32)
    t = (row / float(_SEG - 1)) * 2.0 - 1.0
    acc = b1_ref[...] + inst_ref[0] + t * w1t_ref[0:1, :]
    for k in range(_NUM_FREQ):
        sig = t * (2.0 ** k)
        acc = acc + jnp.sin(sig) * w1t_ref[2 * k + 1:2 * k + 2, :]
        acc = acc + jnp.cos(sig) * w1t_ref[2 * k + 2:2 * k + 3, :]
    out_ref[...] = acc


def _gather_body(table_hbm, idx_hbm, out_hbm, idx_v, buf0, buf1, sem0, sem1):
    wid = lax.axis_index("s") * _NC + lax.axis_index("c")
    base = wid * _BPW
    pltpu.sync_copy(idx_hbm.at[wid], idx_v)
    bufs, sems = (buf0, buf1), (sem0, sem1)
    pltpu.async_copy(table_hbm.at[idx_v.at[0]], buf0, sem0)
    pltpu.async_copy(table_hbm.at[idx_v.at[1]], buf1, sem1)

    @pl.loop(0, _NCHUNK - 2, step=2)
    def _steady(j):
        for b in range(2):
            jj = j + b
            pltpu.make_async_copy(table_hbm.at[idx_v.at[jj]], bufs[b], sems[b]).wait()
            pltpu.sync_copy(bufs[b], out_hbm.at[pl.ds(base + jj * _CH, _CH)])
            pltpu.async_copy(table_hbm.at[idx_v.at[jj + 2]], bufs[b], sems[b])

    for b in range(2):
        jj = _NCHUNK - 2 + b
        pltpu.make_async_copy(table_hbm.at[idx_v.at[jj]], bufs[b], sems[b]).wait()
        pltpu.sync_copy(bufs[b], out_hbm.at[pl.ds(base + jj * _CH, _CH)])


def kernel(frame_id, W1, b1, inst_table):
    w1t = W1.T
    b1_2d = b1.reshape(1, _D)
    inst3 = inst_table.reshape(_NUM_VIDS, 1, _D)

    table = pl.pallas_call(
        _table_body,
        grid=(_NUM_VIDS,),
        in_specs=[
            pl.BlockSpec((_TCH, _D), lambda i: (0, 0)),
            pl.BlockSpec((1, _D), lambda i: (0, 0)),
            pl.BlockSpec((1, 1, _D), lambda i: (i, 0, 0)),
        ],
        out_specs=pl.BlockSpec((_SEG, _D), lambda i: (i, 0)),
        out_shape=jax.ShapeDtypeStruct((_NUM_RAW, _D), jnp.float32),
    )(w1t, b1_2d, inst3)

    idx = frame_id.reshape(_NW, _NCHUNK, _CH)
    mesh = plsc.VectorSubcoreMesh(core_axis_name="c", subcore_axis_name="s",
                                  num_cores=_NC, num_subcores=_NS)
    gather = pl.kernel(
        _gather_body,
        out_type=jax.ShapeDtypeStruct((_B, _D), jnp.float32),
        mesh=mesh,
        scratch_types=[
            pltpu.VMEM((_NCHUNK, _CH), jnp.int32),
            pltpu.VMEM((_CH, _D), jnp.float32),
            pltpu.VMEM((_CH, _D), jnp.float32),
            pltpu.SemaphoreType.DMA,
            pltpu.SemaphoreType.DMA,
        ],
    )
    return gather(table, idx)


# speedup vs baseline: 44.3463x; 2.1515x over previous
import jax
import jax.numpy as jnp
from jax import lax
from jax.experimental import pallas as pl
from jax.experimental.pallas import tpu as pltpu
from jax.experimental.pallas import tpu_sc as plsc

_NUM_FREQ = 6
_NUM_RAW = 16384
_SEG = 2048
_NUM_VIDS = 8
_TCH = 1 + 2 * _NUM_FREQ
_D = 128
_B = 262144
_NC, _NS = 2, 16
_NW = _NC * _NS
_BPW = _B // _NW
_CH = 128
_NCHUNK = _BPW // _CH


def _table_body(wext_ref, out_ref):
    r = lax.broadcasted_iota(jnp.int32, (1, _NUM_RAW), 1)
    vid = lax.shift_right_logical(r, 11)
    frac = (r - lax.shift_left(vid, 11)).astype(jnp.float32)
    t = (frac / float(_SEG - 1)) * 2.0 - 1.0
    rows = [t]
    for k in range(_NUM_FREQ):
        sig = t * (2.0 ** k)
        rows.append(jnp.sin(sig))
        rows.append(jnp.cos(sig))
    rows.append(jnp.ones((1, _NUM_RAW), jnp.float32))
    for v in range(_NUM_VIDS):
        rows.append((vid == v).astype(jnp.float32))
    feats = jnp.concatenate(rows, axis=0)
    out_ref[...] = lax.dot_general(
        feats, wext_ref[...],
        dimension_numbers=(((0,), (1,)), ((), ())),
        preferred_element_type=jnp.float32,
        precision=lax.Precision.HIGHEST,
    )


def _gather_body(table_hbm, idx_hbm, out_hbm, idx_v, buf0, buf1, sem0, sem1):
    wid = lax.axis_index("s") * _NC + lax.axis_index("c")
    base = wid * _BPW
    pltpu.sync_copy(idx_hbm.at[wid], idx_v)
    bufs, sems = (buf0, buf1), (sem0, sem1)
    pltpu.async_copy(table_hbm.at[idx_v.at[0]], buf0, sem0)
    pltpu.async_copy(table_hbm.at[idx_v.at[1]], buf1, sem1)

    @pl.loop(0, _NCHUNK - 2, step=2)
    def _steady(j):
        for b in range(2):
            jj = j + b
            pltpu.make_async_copy(table_hbm.at[idx_v.at[jj]], bufs[b], sems[b]).wait()
            pltpu.sync_copy(bufs[b], out_hbm.at[pl.ds(base + jj * _CH, _CH)])
            pltpu.async_copy(table_hbm.at[idx_v.at[jj + 2]], bufs[b], sems[b])

    for b in range(2):
        jj = _NCHUNK - 2 + b
        pltpu.make_async_copy(table_hbm.at[idx_v.at[jj]], bufs[b], sems[b]).wait()
        pltpu.sync_copy(bufs[b], out_hbm.at[pl.ds(base + jj * _CH, _CH)])


def kernel(frame_id, W1, b1, inst_table):
    wext = jnp.concatenate([W1, b1[:, None], inst_table.T], axis=1)

    table = pl.pallas_call(
        _table_body,
        in_specs=[pl.BlockSpec((_D, _TCH + 1 + _NUM_VIDS), lambda: (0, 0))],
        out_specs=pl.BlockSpec((_NUM_RAW, _D), lambda: (0, 0)),
        out_shape=jax.ShapeDtypeStruct((_NUM_RAW, _D), jnp.float32),
    )(wext)

    idx = frame_id.reshape(_NW, _NCHUNK, _CH)
    mesh = plsc.VectorSubcoreMesh(core_axis_name="c", subcore_axis_name="s",
                                  num_cores=_NC, num_subcores=_NS)
    gather = pl.kernel(
        _gather_body,
        out_type=jax.ShapeDtypeStruct((_B, _D), jnp.float32),
        mesh=mesh,
        scratch_types=[
            pltpu.VMEM((_NCHUNK, _CH), jnp.int32),
            pltpu.VMEM((_CH, _D), jnp.float32),
            pltpu.VMEM((_CH, _D), jnp.float32),
            pltpu.SemaphoreType.DMA,
            pltpu.SemaphoreType.DMA,
        ],
    )
    return gather(table, idx)


# speedup vs baseline: 51.2625x; 1.1560x over previous
import jax
import jax.numpy as jnp
from jax import lax
from jax.experimental import pallas as pl
from jax.experimental.pallas import tpu as pltpu
from jax.experimental.pallas import tpu_sc as plsc

_NUM_FREQ = 6
_NUM_RAW = 16384
_SEG = 2048
_NUM_VIDS = 8
_TCH = 1 + 2 * _NUM_FREQ
_K = 16
_D = 128
_B = 262144
_NC, _NS = 2, 16
_NW = _NC * _NS
_BPW = _B // _NW
_CH = 128
_NCHUNK = _BPW // _CH


def _table_body(wext_ref, inst_ref, out_ref, acc_ref):
    i = pl.program_id(0)

    @pl.when(i == 0)
    def _compute():
        r = lax.broadcasted_iota(jnp.int32, (1, _SEG), 1).astype(jnp.float32)
        t = (r / float(_SEG - 1)) * 2.0 - 1.0
        ri = lax.broadcasted_iota(jnp.int32, (_K, _SEG), 0)
        ci = lax.broadcasted_iota(jnp.int32, (_K, 1), 0)
        kexp = jnp.maximum(lax.shift_right_arithmetic(ci - 1, 1), 0)
        freqs = lax.shift_left(1, kexp).astype(jnp.float32)
        sig = t * freqs
        feats = jnp.where((ri & 1) == 1, jnp.sin(sig), jnp.cos(sig))
        feats = jnp.where(ri == 0, t, feats)
        feats = jnp.where(ri == _TCH, 1.0, feats)
        feats = jnp.where(ri > _TCH, 0.0, feats)
        acc_ref[...] = lax.dot_general(
            feats, wext_ref[...],
            dimension_numbers=(((0,), (1,)), ((), ())),
            preferred_element_type=jnp.float32,
            precision=lax.Precision.HIGHEST,
        )

    out_ref[...] = acc_ref[...] + inst_ref[0]


_NB = 6
_LA = 3
_STEADY_LO = _LA + (_NCHUNK - 2 * _LA) % _NB


def _gather_body(table_hbm, idx_hbm, out_hbm, idx_v, *rest):
    bufs, gsems, osems = rest[:_NB], rest[_NB:2 * _NB], rest[2 * _NB:]
    wid = lax.axis_index("s") * _NC + lax.axis_index("c")
    base = wid * _BPW
    pltpu.sync_copy(idx_hbm.at[wid], idx_v)

    def start_gather(jj, b):
        pltpu.async_copy(table_hbm.at[idx_v.at[jj]], bufs[b], gsems[b])

    def wait_gather(b):
        pltpu.make_async_copy(table_hbm.at[idx_v.at[0]], bufs[b], gsems[b]).wait()

    def start_out(jj, b):
        pltpu.async_copy(bufs[b], out_hbm.at[pl.ds(base + jj * _CH, _CH)], osems[b])

    def wait_out(b):
        pltpu.make_async_copy(bufs[b], out_hbm.at[pl.ds(base, _CH)], osems[b]).wait()

    def full_visit(jj, b):
        wait_gather(b)
        start_out(jj, b)
        s = (b + _LA) % _NB
        wait_out(s)
        start_gather(jj + _LA, s)

    for jj in range(_LA):
        start_gather(jj, jj)
    for jj in range(_LA):
        wait_gather(jj)
        start_out(jj, jj)
        start_gather(jj + _LA, jj + _LA)
    for jj in range(_LA, _STEADY_LO):
        full_visit(jj, jj % _NB)

    @pl.loop(_STEADY_LO, _NCHUNK - _LA, step=_NB)
    def _steady(j):
        for u in range(_NB):
            full_visit(j + u, (_STEADY_LO + u) % _NB)

    for jj in range(_NCHUNK - _LA, _NCHUNK):
        wait_gather(jj % _NB)
        start_out(jj, jj % _NB)
    for b in range(_NB):
        wait_out(b)


def kernel(frame_id, W1, b1, inst_table):
    wext = jnp.concatenate(
        [W1, b1[:, None], jnp.zeros((_D, _K - _TCH - 1), jnp.float32)], axis=1)
    inst3 = inst_table.reshape(_NUM_VIDS, 1, _D)

    table = pl.pallas_call(
        _table_body,
        grid=(_NUM_VIDS,),
        in_specs=[
            pl.BlockSpec((_D, _K), lambda i: (0, 0)),
            pl.BlockSpec((1, 1, _D), lambda i: (i, 0, 0)),
        ],
        out_specs=pl.BlockSpec((_SEG, _D), lambda i: (i, 0)),
        out_shape=jax.ShapeDtypeStruct((_NUM_RAW, _D), jnp.float32),
        scratch_shapes=[pltpu.VMEM((_SEG, _D), jnp.float32)],
    )(wext, inst3)

    idx = frame_id.reshape(_NW, _NCHUNK, _CH)
    mesh = plsc.VectorSubcoreMesh(core_axis_name="c", subcore_axis_name="s",
                                  num_cores=_NC, num_subcores=_NS)
    gather = pl.kernel(
        _gather_body,
        out_type=jax.ShapeDtypeStruct((_B, _D), jnp.float32),
        mesh=mesh,
        scratch_types=(
            [pltpu.VMEM((_NCHUNK, _CH), jnp.int32)]
            + [pltpu.VMEM((_CH, _D), jnp.float32) for _ in range(_NB)]
            + [pltpu.SemaphoreType.DMA for _ in range(2 * _NB)]
        ),
    )
    return gather(table, idx)
